# Initial kernel scaffold; baseline (speedup 1.0000x reference)
import jax
import jax.numpy as jnp
from jax.experimental import pallas as pl


def kernel(a, b, table, W_proj, b_proj, W1, b1, W2, b2):
    raise NotImplementedError("write your pallas kernel here")


# speedup vs baseline: 2.2582x; 2.2582x over previous
import functools

import jax
import jax.numpy as jnp
from jax import lax
from jax.experimental import pallas as pl
from jax.experimental.pallas import tpu as pltpu
from jax.experimental.pallas import tpu_sc as plsc

_EMB = 64
_B = 16384
_L = 50
_PROJ = 128

_NC = 2
_NS = 16
_NW = _NC * _NS

_N_ROWS = 2 * _L * _B
_ROWS_PER_W = _N_ROWS // _NW
_KSUB = 128
_SUBS = 8
_CH = _KSUB * _SUBS
_N_CHUNKS = _ROWS_PER_W // _CH


def _sc_gather(idx, table):
    mesh = plsc.VectorSubcoreMesh(core_axis_name="c", subcore_axis_name="s")

    @functools.partial(
        pl.kernel,
        mesh=mesh,
        out_type=jax.ShapeDtypeStruct((_N_ROWS, _EMB), jnp.float32),
        scratch_types=[
            pltpu.VMEM((_CH,), jnp.int32),
            pltpu.VMEM((_SUBS, _KSUB, _EMB), jnp.float32),
            pltpu.SemaphoreType.DMA,
        ],
        compiler_params=pltpu.CompilerParams(use_tc_tiling_on_sc=False),
    )
    def gather_kernel(idx_hbm, table_hbm, out_hbm, idx_v, rows_v, sem):
        wid = lax.axis_index("s") * _NC + lax.axis_index("c")
        base = wid * _ROWS_PER_W

        def chunk(ci, carry):
            off = base + ci * _CH
            pltpu.sync_copy(idx_hbm.at[pl.ds(off, _CH)], idx_v)
            copies = []
            for j in range(_SUBS):
                copies.append(pltpu.async_copy(
                    table_hbm.at[idx_v.at[pl.ds(j * _KSUB, _KSUB)]],
                    rows_v.at[j], sem))
            for c in copies:
                c.wait()
            for j in range(_SUBS):
                pltpu.sync_copy(rows_v.at[j],
                                out_hbm.at[pl.ds(off + j * _KSUB, _KSUB)])
            return carry

        lax.fori_loop(0, _N_CHUNKS, chunk, 0)

    return gather_kernel(idx, table)


_R = 256
_NB = _B // _R


def _tc_body(ea_ref, eb_ref, wp_ref, bp_ref, w1_ref, b1_ref, w2_ref, b2_ref,
             o_ref):
    wp = wp_ref[...]
    bp = bp_ref[...]

    def tower(ref):
        e = ref[...].reshape(_L * _R, _EMB)
        p = jnp.maximum(
            jnp.dot(e, wp, preferred_element_type=jnp.float32) + bp, 0.0)
        return jnp.mean(p.reshape(_L, _R, _PROJ), axis=0)

    h = jnp.concatenate([tower(ea_ref), tower(eb_ref)], axis=1)
    h1 = jnp.maximum(
        jnp.dot(h, w1_ref[...], preferred_element_type=jnp.float32)
        + b1_ref[...], 0.0)
    o_ref[...] = jnp.sum(h1 * w2_ref[...], axis=1) + b2_ref[0, 0]


def _tc_compute(emb3, W_proj, b_proj2, W1, b1_2, W2r, b2_2):
    return pl.pallas_call(
        _tc_body,
        grid=(_NB,),
        in_specs=[
            pl.BlockSpec((_L, _R, _EMB), lambda i: (0, i, 0)),
            pl.BlockSpec((_L, _R, _EMB), lambda i: (1, i, 0)),
            pl.BlockSpec((_EMB, _PROJ), lambda i: (0, 0)),
            pl.BlockSpec((1, _PROJ), lambda i: (0, 0)),
            pl.BlockSpec((2 * _PROJ, _PROJ), lambda i: (0, 0)),
            pl.BlockSpec((1, _PROJ), lambda i: (0, 0)),
            pl.BlockSpec((1, _PROJ), lambda i: (0, 0)),
            pl.BlockSpec(memory_space=pltpu.SMEM),
        ],
        out_specs=pl.BlockSpec((_R,), lambda i: (i,)),
        out_shape=jax.ShapeDtypeStruct((_B,), jnp.float32),
    )(emb3, emb3, W_proj, b_proj2, W1, b1_2, W2r, b2_2)


def kernel(a, b, table, W_proj, b_proj, W1, b1, W2, b2):
    idx = jnp.concatenate([a.T.reshape(-1), b.T.reshape(-1)])
    emb = _sc_gather(idx, table)
    emb3 = emb.reshape(2 * _L, _B, _EMB)
    return _tc_compute(
        emb3,
        W_proj,
        b_proj.reshape(1, _PROJ),
        W1,
        b1.reshape(1, _PROJ),
        W2.reshape(1, _PROJ),
        b2.reshape(1, 1),
    )
